# Initial kernel scaffold; baseline (speedup 1.0000x reference)
import jax
import jax.numpy as jnp
from jax.experimental import pallas as pl


def kernel(seq, edge_index, edge_weight, samples, fea_w, str_w0, str_w1, W0, b0, a0, W1, b1, a1, g0, bnb0, g1, bnb1):
    raise NotImplementedError("write your pallas kernel here")


# speedup vs baseline: 1.3093x; 1.3093x over previous
import jax
import jax.numpy as jnp
from jax.experimental import pallas as pl

N = 10000
E = 320000
NH = 128
EPS_COS = 1e-8


def _bn(x, g, b):
    m = jnp.mean(x, axis=0)
    v = jnp.var(x, axis=0)
    return (x - m) / jnp.sqrt(v + 1e-5) * g + b


def _prelu(x, a):
    return jnp.where(x >= 0, x, a * x)


def _spmm(edge_index, edge_weight, h):
    dst = edge_index[0]
    src = edge_index[1]
    msg = jnp.take(h, src, axis=0) * edge_weight[:, None]
    return jax.ops.segment_sum(msg, dst, num_segments=N)


def _loss_body(num_ref, hisq_ref, htsq_ref, out_ref):
    num = num_ref[...]
    hisq = hisq_ref[...]
    htsq = htsq_ref[...]
    den = jnp.maximum(jnp.sqrt(hisq) * jnp.sqrt(htsq), EPS_COS)
    sim = num / den
    ex = jnp.exp(sim)
    numerator = ex[:, 0:1]
    denominator = jnp.sum(ex[:, 1:], axis=1, keepdims=True)
    res = -jnp.log(numerator / denominator)
    out_ref[...] = jnp.sum(res, axis=0, keepdims=True) / N


def kernel(seq, edge_index, edge_weight, samples, fea_w, str_w0, str_w1,
           W0, b0, a0, W1, b1, a1, g0, bnb0, g1, bnb1):
    x = jnp.squeeze(seq, axis=0)

    W0cat = jnp.concatenate([fea_w[0][:, None] * W0, W0], axis=1)
    h1 = x @ W0cat
    o1 = _spmm(edge_index, edge_weight, h1) + jnp.tile(b0, 2)
    o1 = _prelu(o1, a0)
    prompt1 = jnp.concatenate([jnp.ones((NH,), o1.dtype), str_w0[0]])
    o1 = o1 * prompt1
    o1 = _bn(o1, jnp.tile(g0, 2), jnp.tile(bnb0, 2))

    h2 = jnp.concatenate([o1[:, :NH] @ W1, o1[:, NH:] @ W1], axis=1)
    o2 = _spmm(edge_index, edge_weight, h2) + jnp.tile(b1, 2)
    o2 = _prelu(o2, a1)
    prompt2 = jnp.concatenate([jnp.ones((NH,), o2.dtype), str_w1[0]])
    o2 = o2 * prompt2
    o2 = _bn(o2, jnp.tile(g1, 2), jnp.tile(bnb1, 2))

    logits = jax.nn.elu(o2[:, :NH]) + jax.nn.elu(o2[:, NH:])

    h_t = jnp.take(logits, samples, axis=0)
    num = jnp.einsum("nd,nkd->nk", logits, h_t)
    hisq = jnp.sum(logits * logits, axis=1, keepdims=True)
    htsq = jnp.sum(h_t * h_t, axis=2)

    out = pl.pallas_call(
        _loss_body,
        out_shape=jax.ShapeDtypeStruct((1, 1), jnp.float32),
    )(num, hisq, htsq)
    return out[0, 0]


# speedup vs baseline: 4.9230x; 3.7600x over previous
import functools

import jax
import jax.numpy as jnp
from jax import lax
from jax.experimental import pallas as pl
from jax.experimental.pallas import tpu as pltpu
from jax.experimental.pallas import tpu_sc as plsc

N = 10000
E = 320000
NH = 128
EPS_COS = 1e-8

NC, NS = 2, 16
NW = NC * NS

EPW = E // NS
S_CHUNK = 80
S_NCH = EPW // S_CHUNK
NP = 10240
ROWS_PW = NP // NS
ZR = 160
HW = 64


def _scale_chunk(rows_ref, wb_ref, hw):
    def edge(i, _):
        wv = wb_ref[i, :]
        for j in range(hw):
            sl = pl.ds(j * 16, 16)
            rows_ref[i, sl] = rows_ref[i, sl] * wv
        return 0

    lax.fori_loop(0, S_CHUNK, edge, 0, unroll=4)


def _sc_spmm_body(h_hbm, dst_hbm, src_hbm, wbc_hbm, out_hbm,
                  dst_all, src_all, idx0, idx1, dc0, dc1,
                  rows0, rows1, wb0, wb1, zbuf, acc, sem0, sem1, lsem):
    c = lax.axis_index("c")
    s = lax.axis_index("s")
    ebase = s * EPW

    pltpu.async_copy(dst_hbm.at[pl.ds(ebase, EPW)], dst_all, lsem).wait()
    pltpu.async_copy(src_hbm.at[pl.ds(ebase, EPW)], src_all, lsem).wait()

    idx_bufs = (idx0, idx1)
    dc_bufs = (dc0, dc1)
    row_bufs = (rows0, rows1)
    wb_bufs = (wb0, wb1)
    sems = (sem0, sem1)

    for hh in range(2):
        def zrow(r, _):
            for j in range(HW // 16):
                zbuf[r, pl.ds(j * 16, 16)] = jnp.zeros((16,), jnp.float32)
            return 0

        lax.fori_loop(0, ZR, zrow, 0, unroll=4)
        for k in range(ROWS_PW // ZR):
            pltpu.sync_copy(zbuf, acc.at[pl.ds(s * ROWS_PW + k * ZR, ZR)])
        plsc.subcore_barrier()

        rbase = 2 * c * N + hh

        def prep_and_fire(t, b):
            off = t * S_CHUNK
            for g in range(S_CHUNK // 16):
                sl = pl.ds(g * 16, 16)
                idx_bufs[b][sl] = src_all[pl.ds(off + g * 16, 16)] * 2 + rbase
                dc_bufs[b][sl] = dst_all[pl.ds(off + g * 16, 16)]
            pltpu.async_copy(wbc_hbm.at[pl.ds(ebase + off, S_CHUNK)],
                             wb_bufs[b], sems[b])
            pltpu.async_copy(h_hbm.at[idx_bufs[b]], row_bufs[b], sems[b])

        prep_and_fire(0, 0)
        prep_and_fire(1, 1)

        def pair(i, _):
            t2 = i * 2
            for b in range(2):
                t = t2 + b
                pltpu.make_async_copy(wbc_hbm.at[pl.ds(0, S_CHUNK)],
                                      wb_bufs[b], sems[b]).wait()
                pltpu.make_async_copy(h_hbm.at[idx_bufs[b]], row_bufs[b],
                                      sems[b]).wait()
                _scale_chunk(row_bufs[b], wb_bufs[b], HW // 16)
                pltpu.sync_copy(row_bufs[b], acc.at[dc_bufs[b]], add=True)

                @pl.when(t + 2 < S_NCH)
                def _():
                    prep_and_fire(t + 2, b)

            return 0

        lax.fori_loop(0, S_NCH // 2, pair, 0)

        plsc.subcore_barrier()
        obase = (2 * c + hh) * NP
        for k in range(ROWS_PW // ZR):
            r0 = s * ROWS_PW + k * ZR
            pltpu.sync_copy(acc.at[pl.ds(r0, ZR)],
                            out_hbm.at[pl.ds(obase + r0, ZR)])
        plsc.subcore_barrier()


def _sc_spmm(h64, dst, src, wbc):
    mesh = plsc.VectorSubcoreMesh(core_axis_name="c", subcore_axis_name="s")
    return pl.kernel(
        _sc_spmm_body,
        out_type=jax.ShapeDtypeStruct((4 * NP, HW), jnp.float32),
        mesh=mesh,
        compiler_params=pltpu.CompilerParams(use_tc_tiling_on_sc=False),
        scratch_types=[
            pltpu.VMEM((EPW,), jnp.int32),
            pltpu.VMEM((EPW,), jnp.int32),
            pltpu.VMEM((S_CHUNK,), jnp.int32),
            pltpu.VMEM((S_CHUNK,), jnp.int32),
            pltpu.VMEM((S_CHUNK,), jnp.int32),
            pltpu.VMEM((S_CHUNK,), jnp.int32),
            pltpu.VMEM((S_CHUNK, HW), jnp.float32),
            pltpu.VMEM((S_CHUNK, HW), jnp.float32),
            pltpu.VMEM((S_CHUNK, 16), jnp.float32),
            pltpu.VMEM((S_CHUNK, 16), jnp.float32),
            pltpu.VMEM((ZR, HW), jnp.float32),
            pltpu.VMEM_SHARED((NP, HW), jnp.float32),
            pltpu.SemaphoreType.DMA,
            pltpu.SemaphoreType.DMA,
            pltpu.SemaphoreType.DMA,
        ],
    )(h64, dst, src, wbc)


G_TOT = 50176
G_PER_W = G_TOT // NW
G_CHUNK = 112
G_NCH = G_PER_W // G_CHUNK


def _sc_gather_body(table_hbm, idx_hbm, out_hbm, idx_v, rows_v, sem):
    wid = lax.axis_index("s") * NC + lax.axis_index("c")
    base0 = wid * G_PER_W

    def chunk(t, _):
        base = base0 + t * G_CHUNK
        pltpu.sync_copy(idx_hbm.at[pl.ds(base, G_CHUNK)], idx_v)
        pltpu.async_copy(table_hbm.at[idx_v], rows_v, sem).wait()
        pltpu.sync_copy(rows_v, out_hbm.at[pl.ds(base, G_CHUNK)])
        return 0

    lax.fori_loop(0, G_NCH, chunk, 0)


def _sc_gather_rows(table, idx_padded):
    mesh = plsc.VectorSubcoreMesh(core_axis_name="c", subcore_axis_name="s")
    return pl.kernel(
        _sc_gather_body,
        out_type=jax.ShapeDtypeStruct((G_TOT, NH), jnp.float32),
        mesh=mesh,
        scratch_types=[
            pltpu.VMEM((G_CHUNK,), jnp.int32),
            pltpu.VMEM((G_CHUNK, NH), jnp.float32),
            pltpu.SemaphoreType.DMA,
        ],
    )(table, idx_padded)


def _unsplit(out64):
    o = out64.reshape(2, 2, NP, HW)[:, :, :N, :]
    return jnp.concatenate([o[:, 0], o[:, 1]], axis=-1)


def _bn2(o, g, b):
    m = jnp.mean(o, axis=1, keepdims=True)
    v = jnp.var(o, axis=1, keepdims=True)
    return (o - m) / jnp.sqrt(v + 1e-5) * g + b


def _prelu(x, a):
    return jnp.where(x >= 0, x, a * x)


def _loss_body(num_ref, hisq_ref, htsq_ref, out_ref):
    num = num_ref[...]
    hisq = hisq_ref[...]
    htsq = htsq_ref[...]
    den = jnp.maximum(jnp.sqrt(hisq) * jnp.sqrt(htsq), EPS_COS)
    sim = num / den
    ex = jnp.exp(sim)
    numerator = ex[:, 0:1]
    denominator = jnp.sum(ex[:, 1:], axis=1, keepdims=True)
    res = -jnp.log(numerator / denominator)
    out_ref[...] = jnp.sum(res, axis=0, keepdims=True) / N


def kernel(seq, edge_index, edge_weight, samples, fea_w, str_w0, str_w1,
           W0, b0, a0, W1, b1, a1, g0, bnb0, g1, bnb1):
    x = jnp.squeeze(seq, axis=0)
    dst = edge_index[0]
    src = edge_index[1]
    wbc = jnp.broadcast_to(edge_weight[:, None], (E, 16))

    W0pair = jnp.stack([fea_w[0][:, None] * W0, W0])
    h1 = jnp.einsum("nk,pkj->pnj", x, W0pair).reshape(4 * N, HW)
    o1 = _unsplit(_sc_spmm(h1, dst, src, wbc))
    o1 = _prelu(o1 + b0, a0)
    o1 = o1 * jnp.stack([jnp.ones((NH,), o1.dtype), str_w0[0]])[:, None, :]
    o1 = _bn2(o1, g0, bnb0)

    h2 = jnp.einsum("pnj,jk->pnk", o1, W1).reshape(4 * N, HW)
    o2 = _unsplit(_sc_spmm(h2, dst, src, wbc))
    o2 = _prelu(o2 + b1, a1)
    o2 = o2 * jnp.stack([jnp.ones((NH,), o2.dtype), str_w1[0]])[:, None, :]
    o2 = _bn2(o2, g1, bnb1)

    logits = jax.nn.elu(o2[0]) + jax.nn.elu(o2[1])

    idx_flat = jnp.pad(samples.reshape(-1), (0, G_TOT - 5 * N))
    h_t = _sc_gather_rows(logits, idx_flat)[: 5 * N].reshape(N, 5, NH)
    num = jnp.einsum("nd,nkd->nk", logits, h_t)
    hisq = jnp.sum(logits * logits, axis=1, keepdims=True)
    htsq = jnp.sum(h_t * h_t, axis=2)

    out = pl.pallas_call(
        _loss_body,
        out_shape=jax.ShapeDtypeStruct((1, 1), jnp.float32),
    )(num, hisq, htsq)
    return out[0, 0]


# speedup vs baseline: 5.4043x; 1.0978x over previous
import functools

import jax
import jax.numpy as jnp
from jax import lax
from jax.experimental import pallas as pl
from jax.experimental.pallas import tpu as pltpu
from jax.experimental.pallas import tpu_sc as plsc

N = 10000
E = 320000
NH = 128
EPS_COS = 1e-8

NC, NS = 2, 16
NW = NC * NS

EPW = E // NS
S_CHUNK = 80
S_NCH = EPW // S_CHUNK
NP = 10240
ROWS_PW = NP // NS
ZR = 160
HW = 64


def _scale_chunk(rows_ref, wb_ref):
    def edge(i, _):
        wv = wb_ref[i, :]
        for j in range(HW // 16):
            sl = pl.ds(j * 16, 16)
            rows_ref[i, sl] = rows_ref[i, sl] * wv
        return 0

    lax.fori_loop(0, S_CHUNK, edge, 0, unroll=8)


NBUF = 5


def _sc_spmm_body(h_hbm, dst_hbm, src_hbm, wbc_hbm, out_hbm,
                  dst_all, src_all, idx_bufs, dc_bufs, rows_bufs, wb_bufs,
                  zbuf, acc, gsems, ssems, lsem):
    c = lax.axis_index("c")
    s = lax.axis_index("s")
    ebase = s * EPW

    pltpu.async_copy(dst_hbm.at[pl.ds(ebase, EPW)], dst_all, lsem).wait()
    pltpu.async_copy(src_hbm.at[pl.ds(ebase, EPW)], src_all, lsem).wait()

    for hh in range(2):
        def zrow(r, _):
            for j in range(HW // 16):
                zbuf[r, pl.ds(j * 16, 16)] = jnp.zeros((16,), jnp.float32)
            return 0

        lax.fori_loop(0, ZR, zrow, 0, unroll=4)
        for k in range(ROWS_PW // ZR):
            pltpu.sync_copy(zbuf, acc.at[pl.ds(s * ROWS_PW + k * ZR, ZR)])
        plsc.subcore_barrier()

        rbase = 2 * c * N + hh

        def prep_and_fire(t, b):
            off = t * S_CHUNK
            for g in range(S_CHUNK // 16):
                sl = pl.ds(g * 16, 16)
                idx_bufs[b][sl] = src_all[pl.ds(off + g * 16, 16)] * 2 + rbase
                dc_bufs[b][sl] = dst_all[pl.ds(off + g * 16, 16)]
            pltpu.async_copy(wbc_hbm.at[pl.ds(ebase + off, S_CHUNK)],
                             wb_bufs[b], gsems[b])
            pltpu.async_copy(h_hbm.at[idx_bufs[b]], rows_bufs[b], gsems[b])

        prep_and_fire(0, 0)
        prep_and_fire(1, 1)

        def quint(q, _):
            v0 = q * NBUF
            for b in range(NBUF):
                v = v0 + b
                pltpu.make_async_copy(wbc_hbm.at[pl.ds(0, S_CHUNK)],
                                      wb_bufs[b], gsems[b]).wait()
                pltpu.make_async_copy(h_hbm.at[idx_bufs[b]], rows_bufs[b],
                                      gsems[b]).wait()
                _scale_chunk(rows_bufs[b], wb_bufs[b])
                pltpu.async_copy(rows_bufs[b], acc.at[dc_bufs[b]], ssems[b],
                                 add=True)
                bp = (b + 2) % NBUF

                @pl.when(v + 2 < S_NCH)
                def _():
                    @pl.when(v >= 3)
                    def _():
                        pltpu.make_async_copy(rows_bufs[bp], acc.at[dc_bufs[bp]],
                                              ssems[bp]).wait()

                    prep_and_fire(v + 2, bp)

            return 0

        lax.fori_loop(0, S_NCH // NBUF, quint, 0)

        for b in range(NBUF):
            pltpu.make_async_copy(rows_bufs[b], acc.at[dc_bufs[b]],
                                  ssems[b]).wait()

        plsc.subcore_barrier()
        obase = (2 * c + hh) * NP
        for k in range(ROWS_PW // ZR):
            r0 = s * ROWS_PW + k * ZR
            pltpu.sync_copy(acc.at[pl.ds(r0, ZR)],
                            out_hbm.at[pl.ds(obase + r0, ZR)])
        plsc.subcore_barrier()


def _sc_spmm(h64, dst, src, wbc):
    mesh = plsc.VectorSubcoreMesh(core_axis_name="c", subcore_axis_name="s")
    return pl.kernel(
        _sc_spmm_body,
        out_type=jax.ShapeDtypeStruct((4 * NP, HW), jnp.float32),
        mesh=mesh,
        compiler_params=pltpu.CompilerParams(use_tc_tiling_on_sc=False),
        scratch_types=[
            pltpu.VMEM((EPW,), jnp.int32),
            pltpu.VMEM((EPW,), jnp.int32),
            [pltpu.VMEM((S_CHUNK,), jnp.int32) for _ in range(NBUF)],
            [pltpu.VMEM((S_CHUNK,), jnp.int32) for _ in range(NBUF)],
            [pltpu.VMEM((S_CHUNK, HW), jnp.float32) for _ in range(NBUF)],
            [pltpu.VMEM((S_CHUNK, 16), jnp.float32) for _ in range(NBUF)],
            pltpu.VMEM((ZR, HW), jnp.float32),
            pltpu.VMEM_SHARED((NP, HW), jnp.float32),
            [pltpu.SemaphoreType.DMA for _ in range(NBUF)],
            [pltpu.SemaphoreType.DMA for _ in range(NBUF)],
            pltpu.SemaphoreType.DMA,
        ],
    )(h64, dst, src, wbc)


G_TOT = 50176
G_PER_W = G_TOT // NW
G_CHUNK = 112
G_NCH = G_PER_W // G_CHUNK


def _sc_gather_body(table_hbm, idx_hbm, out_hbm, idx_v, rows_v, sem):
    wid = lax.axis_index("s") * NC + lax.axis_index("c")
    base0 = wid * G_PER_W

    def chunk(t, _):
        base = base0 + t * G_CHUNK
        pltpu.sync_copy(idx_hbm.at[pl.ds(base, G_CHUNK)], idx_v)
        pltpu.async_copy(table_hbm.at[idx_v], rows_v, sem).wait()
        pltpu.sync_copy(rows_v, out_hbm.at[pl.ds(base, G_CHUNK)])
        return 0

    lax.fori_loop(0, G_NCH, chunk, 0)


def _sc_gather_rows(table, idx_padded):
    mesh = plsc.VectorSubcoreMesh(core_axis_name="c", subcore_axis_name="s")
    return pl.kernel(
        _sc_gather_body,
        out_type=jax.ShapeDtypeStruct((G_TOT, NH), jnp.float32),
        mesh=mesh,
        scratch_types=[
            pltpu.VMEM((G_CHUNK,), jnp.int32),
            pltpu.VMEM((G_CHUNK, NH), jnp.float32),
            pltpu.SemaphoreType.DMA,
        ],
    )(table, idx_padded)


def _unsplit(out64):
    o = out64.reshape(2, 2, NP, HW)[:, :, :N, :]
    return jnp.concatenate([o[:, 0], o[:, 1]], axis=-1)


def _bn2(o, g, b):
    m = jnp.mean(o, axis=1, keepdims=True)
    v = jnp.var(o, axis=1, keepdims=True)
    return (o - m) / jnp.sqrt(v + 1e-5) * g + b


def _prelu(x, a):
    return jnp.where(x >= 0, x, a * x)


def _loss_body(num_ref, hisq_ref, htsq_ref, out_ref):
    num = num_ref[...]
    hisq = hisq_ref[...]
    htsq = htsq_ref[...]
    den = jnp.maximum(jnp.sqrt(hisq) * jnp.sqrt(htsq), EPS_COS)
    sim = num / den
    ex = jnp.exp(sim)
    numerator = ex[:, 0:1]
    denominator = jnp.sum(ex[:, 1:], axis=1, keepdims=True)
    res = -jnp.log(numerator / denominator)
    out_ref[...] = jnp.sum(res, axis=0, keepdims=True) / N


def kernel(seq, edge_index, edge_weight, samples, fea_w, str_w0, str_w1,
           W0, b0, a0, W1, b1, a1, g0, bnb0, g1, bnb1):
    x = jnp.squeeze(seq, axis=0)
    dst = edge_index[0]
    src = edge_index[1]
    wbc = jnp.broadcast_to(edge_weight[:, None], (E, 16))

    W0pair = jnp.stack([fea_w[0][:, None] * W0, W0])
    h1 = jnp.einsum("nk,pkj->pnj", x, W0pair).reshape(4 * N, HW)
    o1 = _unsplit(_sc_spmm(h1, dst, src, wbc))
    o1 = _prelu(o1 + b0, a0)
    o1 = o1 * jnp.stack([jnp.ones((NH,), o1.dtype), str_w0[0]])[:, None, :]
    o1 = _bn2(o1, g0, bnb0)

    h2 = jnp.einsum("pnj,jk->pnk", o1, W1).reshape(4 * N, HW)
    o2 = _unsplit(_sc_spmm(h2, dst, src, wbc))
    o2 = _prelu(o2 + b1, a1)
    o2 = o2 * jnp.stack([jnp.ones((NH,), o2.dtype), str_w1[0]])[:, None, :]
    o2 = _bn2(o2, g1, bnb1)

    logits = jax.nn.elu(o2[0]) + jax.nn.elu(o2[1])

    idx_flat = jnp.pad(samples.reshape(-1), (0, G_TOT - 5 * N))
    h_t = _sc_gather_rows(logits, idx_flat)[: 5 * N].reshape(N, 5, NH)
    num = jnp.einsum("nd,nkd->nk", logits, h_t)
    hisq = jnp.sum(logits * logits, axis=1, keepdims=True)
    htsq = jnp.sum(h_t * h_t, axis=2)

    out = pl.pallas_call(
        _loss_body,
        out_shape=jax.ShapeDtypeStruct((1, 1), jnp.float32),
    )(num, hisq, htsq)
    return out[0, 0]
